# Initial kernel scaffold; baseline (speedup 1.0000x reference)
import jax
import jax.numpy as jnp
from jax.experimental import pallas as pl


def kernel(root_vecs, tree_vecs, inter_vecs, graph_vecs, tree_segment_ids, anchor_idx, anchor_segment_ids, W_T, b_T, W_G, b_G):
    raise NotImplementedError("write your pallas kernel here")


# speedup vs baseline: 4.7676x; 4.7676x over previous
import functools

import jax
import jax.numpy as jnp
from jax import lax
from jax.experimental import pallas as pl
from jax.experimental.pallas import tpu as pltpu
from jax.experimental.pallas import tpu_sc as plsc

B = 4096
H = 128
L = 64

NC = 2
NS = 16
NW = NC * NS
BLK = 128
DUMMY = 128
R = B + DUMMY

_MESH = dict(core_axis_name="c", subcore_axis_name="s", num_cores=NC,
             num_subcores=NS)


def _fill_rows(ref, value):
    width = ref.shape[1]
    v16 = jnp.full((16,), value, jnp.float32)

    @pl.loop(0, BLK)
    def _fill(i):
        for h in range(width // 16):
            ref[i, pl.ds(h * 16, 16)] = v16


def _zero_acc(acc, zbuf, sid):
    n_blocks = R // BLK
    for k in range((n_blocks + NS - 1) // NS):
        blk = k * NS + sid

        def _do():
            pltpu.sync_copy(zbuf, acc.at[pl.ds(blk * BLK, BLK)])

        if (k + 1) * NS <= n_blocks:
            _do()
        else:
            pl.when(blk < n_blocks)(_do)


def _writeout(acc, out_ref, cid, sid):
    for k in range(B // BLK // NS):
        row0 = (k * NS + sid) * BLK
        pltpu.sync_copy(acc.at[pl.ds(row0, BLK)],
                        out_ref.at[cid, pl.ds(row0, BLK)])


def _sc_pool(tree_vecs, inter_vecs, graph_vecs, tseg3d, aidx3d, aseg3d,
             n_nodes, n_anchors, tb_per_w, ab_per_w):
    N = n_nodes
    A = n_anchors
    tail_t = N % BLK
    mesh = plsc.VectorSubcoreMesh(**_MESH)

    @functools.partial(
        pl.kernel,
        mesh=mesh,
        out_type=[
            jax.ShapeDtypeStruct((NC, B, H), jnp.float32),
            jax.ShapeDtypeStruct((NC, B, H), jnp.float32),
            jax.ShapeDtypeStruct((NC, B, H), jnp.float32),
        ],
        scratch_types=[
            pltpu.VMEM((tb_per_w, BLK), jnp.int32),
            pltpu.VMEM((ab_per_w, BLK), jnp.int32),
            pltpu.VMEM((ab_per_w, BLK), jnp.int32),
            pltpu.VMEM((BLK, H), jnp.float32),
            pltpu.VMEM((BLK, H), jnp.float32),
            pltpu.VMEM((BLK, H), jnp.float32),
            pltpu.VMEM_SHARED((R, H), jnp.float32),
            pltpu.SemaphoreType.DMA,
        ],
    )
    def body(tree_hbm, inter_hbm, graph_hbm, tseg_hbm, aidx_hbm, aseg_hbm,
             out_t, out_i, out_g,
             ids_v, ai_v, as_v, buf0, buf1, zbuf, acc, sem):
        cid = lax.axis_index("c")
        sid = lax.axis_index("s")
        wid = sid * NC + cid

        _fill_rows(zbuf, 0.0)

        pltpu.sync_copy(tseg_hbm.at[wid], ids_v)
        pltpu.sync_copy(aidx_hbm.at[wid], ai_v)
        pltpu.sync_copy(aseg_hbm.at[wid], as_v)

        def seg_sum_phase(src_hbm):
            for j in range(tb_per_w):
                start = (wid * tb_per_w + j) * BLK
                buf = buf0 if j % 2 == 0 else buf1

                @pl.when(start + BLK <= N)
                def _full():
                    pltpu.sync_copy(src_hbm.at[pl.ds(start, BLK)], buf)
                    pltpu.sync_copy(buf, acc.at[ids_v.at[j]], add=True)

                if tail_t:
                    @pl.when(jnp.logical_and(start < N, start + BLK > N))
                    def _partial():
                        pltpu.sync_copy(src_hbm.at[pl.ds(start, tail_t)],
                                        buf.at[pl.ds(0, tail_t)])
                        pltpu.sync_copy(buf, acc.at[ids_v.at[j]], add=True)

        _zero_acc(acc, zbuf, sid)
        plsc.subcore_barrier()
        seg_sum_phase(tree_hbm)
        plsc.subcore_barrier()
        _writeout(acc, out_t, cid, sid)
        plsc.subcore_barrier()

        _zero_acc(acc, zbuf, sid)
        plsc.subcore_barrier()
        seg_sum_phase(inter_hbm)
        plsc.subcore_barrier()
        _writeout(acc, out_i, cid, sid)
        plsc.subcore_barrier()

        _zero_acc(acc, zbuf, sid)
        plsc.subcore_barrier()
        for j in range(ab_per_w):
            astart = (wid * ab_per_w + j) * BLK
            buf = buf0 if j % 2 == 0 else buf1

            @pl.when(astart < A)
            def _anchors():
                pltpu.async_copy(graph_hbm.at[ai_v.at[j]], buf, sem).wait()
                pltpu.sync_copy(buf, acc.at[as_v.at[j]], add=True)
        plsc.subcore_barrier()
        _writeout(acc, out_g, cid, sid)

    return body(tree_vecs, inter_vecs, graph_vecs, tseg3d, aidx3d, aseg3d)


def _sc_counts(tseg3d, n_nodes, tb_per_w):
    N = n_nodes
    mesh = plsc.VectorSubcoreMesh(**_MESH)

    @functools.partial(
        pl.kernel,
        mesh=mesh,
        out_type=jax.ShapeDtypeStruct((NC, B, H), jnp.float32),
        scratch_types=[
            pltpu.VMEM((tb_per_w, BLK), jnp.int32),
            pltpu.VMEM((BLK, H), jnp.float32),
            pltpu.VMEM((BLK, H), jnp.float32),
            pltpu.VMEM_SHARED((R, H), jnp.float32),
        ],
    )
    def body(tseg_hbm, out_c, ids_v, ones_v, zbuf, acc):
        cid = lax.axis_index("c")
        sid = lax.axis_index("s")
        wid = sid * NC + cid

        _fill_rows(zbuf, 0.0)
        _fill_rows(ones_v, 1.0)
        pltpu.sync_copy(tseg_hbm.at[wid], ids_v)

        _zero_acc(acc, zbuf, sid)
        plsc.subcore_barrier()
        for j in range(tb_per_w):
            start = (wid * tb_per_w + j) * BLK

            @pl.when(start < N)
            def _blk():
                pltpu.sync_copy(ones_v, acc.at[ids_v.at[j]], add=True)
        plsc.subcore_barrier()
        _writeout(acc, out_c, cid, sid)

    return body(tseg3d)


def _tc_heads_body(root_ref, t_ref, i_ref, g_ref, c_ref,
                   wt_ref, bt_ref, wg_ref, bg_ref, out_t_ref, out_g_ref):
    t = t_ref[0] + t_ref[1]
    im = i_ref[0] + i_ref[1]
    g = g_ref[0] + g_ref[1]
    c = c_ref[0, :, 0:1] + c_ref[1, :, 0:1]
    inv = 1.0 / jnp.maximum(c, 1.0)
    wt = wt_ref[...]
    dot = functools.partial(jnp.dot, preferred_element_type=jnp.float32)
    out = dot(root_ref[...], wt[0:H])
    out += dot(t * inv, wt[H:2 * H])
    out += dot(im * inv, wt[2 * H:3 * H])
    out_t_ref[...] = out + bt_ref[...]
    out_g_ref[...] = dot(g, wg_ref[...]) + bg_ref[...]


def _tc_heads(root_vecs, t_part, i_part, g_part, c_part, W_T, b_T, W_G, b_G):
    bsz = 512
    grid = (B // bsz,)
    return pl.pallas_call(
        _tc_heads_body,
        grid=grid,
        in_specs=[
            pl.BlockSpec((bsz, H), lambda i: (i, 0)),
            pl.BlockSpec((NC, bsz, H), lambda i: (0, i, 0)),
            pl.BlockSpec((NC, bsz, H), lambda i: (0, i, 0)),
            pl.BlockSpec((NC, bsz, H), lambda i: (0, i, 0)),
            pl.BlockSpec((NC, bsz, H), lambda i: (0, i, 0)),
            pl.BlockSpec((3 * H, L), lambda i: (0, 0)),
            pl.BlockSpec((1, L), lambda i: (0, 0)),
            pl.BlockSpec((H, L), lambda i: (0, 0)),
            pl.BlockSpec((1, L), lambda i: (0, 0)),
        ],
        out_specs=[
            pl.BlockSpec((bsz, L), lambda i: (i, 0)),
            pl.BlockSpec((bsz, L), lambda i: (i, 0)),
        ],
        out_shape=[
            jax.ShapeDtypeStruct((B, L), jnp.float32),
            jax.ShapeDtypeStruct((B, L), jnp.float32),
        ],
    )(root_vecs, t_part, i_part, g_part, c_part, W_T, b_T, W_G, b_G)


def kernel(root_vecs, tree_vecs, inter_vecs, graph_vecs, tree_segment_ids,
           anchor_idx, anchor_segment_ids, W_T, b_T, W_G, b_G):
    N = tree_vecs.shape[0]
    A = anchor_idx.shape[0]

    tb_per_w = -(-N // (NW * BLK))
    ab_per_w = -(-A // (NW * BLK))
    n_pad = NW * tb_per_w * BLK - N
    a_pad = NW * ab_per_w * BLK - A

    tseg = tree_segment_ids.astype(jnp.int32)
    tseg = jnp.concatenate(
        [tseg, B + (jnp.arange(n_pad, dtype=jnp.int32) % DUMMY)])
    tseg3d = tseg.reshape(NW, tb_per_w, BLK)

    aidx = anchor_idx.astype(jnp.int32)
    aidx = jnp.concatenate(
        [aidx, (jnp.arange(a_pad, dtype=jnp.int32) * 997) % N])
    aidx3d = aidx.reshape(NW, ab_per_w, BLK)

    aseg = anchor_segment_ids.astype(jnp.int32)
    aseg = jnp.concatenate(
        [aseg, B + (jnp.arange(a_pad, dtype=jnp.int32) % DUMMY)])
    aseg3d = aseg.reshape(NW, ab_per_w, BLK)

    t_part, i_part, g_part = _sc_pool(
        tree_vecs, inter_vecs, graph_vecs, tseg3d, aidx3d, aseg3d,
        N, A, tb_per_w, ab_per_w)
    c_part = _sc_counts(tseg3d, N, tb_per_w)

    tree_out, graph_out = _tc_heads(
        root_vecs, t_part, i_part, g_part, c_part,
        W_T, b_T.reshape(1, L), W_G, b_G.reshape(1, L))
    return (tree_out, tree_out, graph_out)


# speedup vs baseline: 6.2009x; 1.3006x over previous
import functools

import jax
import jax.numpy as jnp
from jax import lax
from jax.experimental import pallas as pl
from jax.experimental.pallas import tpu as pltpu
from jax.experimental.pallas import tpu_sc as plsc

B = 4096
H = 128
L = 64

NC = 2
NS = 16
NW = NC * NS
BLK = 128
DUMMY = 128
R = B + DUMMY

_MESH = dict(core_axis_name="c", subcore_axis_name="s", num_cores=NC,
             num_subcores=NS)


def _fill_rows(ref, value):
    width = ref.shape[1]
    v16 = jnp.full((16,), value, jnp.float32)

    @pl.loop(0, BLK)
    def _fill(i):
        for h in range(width // 16):
            ref[i, pl.ds(h * 16, 16)] = v16


def _zero_acc(acc, zbuf, sid):
    n_blocks = R // BLK
    for k in range((n_blocks + NS - 1) // NS):
        blk = k * NS + sid

        def _do():
            pltpu.sync_copy(zbuf, acc.at[pl.ds(blk * BLK, BLK)])

        if (k + 1) * NS <= n_blocks:
            _do()
        else:
            pl.when(blk < n_blocks)(_do)


def _writeout(acc, out_ref, cid, sid):
    for k in range(B // BLK // NS):
        row0 = (k * NS + sid) * BLK
        pltpu.sync_copy(acc.at[pl.ds(row0, BLK)],
                        out_ref.at[cid, pl.ds(row0, BLK)])


def _sc_pool(tree_vecs, inter_vecs, graph_vecs, tseg3d, aidx3d, aseg3d,
             n_nodes, n_anchors, tb_per_w, ab_per_w):
    N = n_nodes
    A = n_anchors
    tail_t = N % BLK
    mesh = plsc.VectorSubcoreMesh(**_MESH)

    @functools.partial(
        pl.kernel,
        mesh=mesh,
        out_type=[
            jax.ShapeDtypeStruct((NC, B, H), jnp.float32),
            jax.ShapeDtypeStruct((NC, B, H), jnp.float32),
            jax.ShapeDtypeStruct((NC, B, H), jnp.float32),
        ],
        scratch_types=[
            pltpu.VMEM((tb_per_w, BLK), jnp.int32),
            pltpu.VMEM((ab_per_w, BLK), jnp.int32),
            pltpu.VMEM((ab_per_w, BLK), jnp.int32),
            pltpu.VMEM((BLK, H), jnp.float32),
            pltpu.VMEM((BLK, H), jnp.float32),
            pltpu.VMEM((BLK, H), jnp.float32),
            pltpu.VMEM_SHARED((R, H), jnp.float32),
            pltpu.SemaphoreType.DMA,
            pltpu.SemaphoreType.DMA,
        ],
    )
    def body(tree_hbm, inter_hbm, graph_hbm, tseg_hbm, aidx_hbm, aseg_hbm,
             out_t, out_i, out_g,
             ids_v, ai_v, as_v, buf0, buf1, zbuf, acc, sem0, sem1):
        cid = lax.axis_index("c")
        sid = lax.axis_index("s")
        wid = sid * NC + cid

        _fill_rows(zbuf, 0.0)

        pltpu.sync_copy(tseg_hbm.at[wid], ids_v)
        pltpu.sync_copy(aidx_hbm.at[wid], ai_v)
        pltpu.sync_copy(aseg_hbm.at[wid], as_v)

        bufs = (buf0, buf1)
        sems = (sem0, sem1)

        def seg_sum_phase(src_hbm):
            def descs(j):
                start = (wid * tb_per_w + j) * BLK
                buf, sem = bufs[j % 2], sems[j % 2]
                full = pltpu.make_async_copy(
                    src_hbm.at[pl.ds(start, BLK)], buf, sem)
                part = pltpu.make_async_copy(
                    src_hbm.at[pl.ds(start, tail_t)],
                    buf.at[pl.ds(0, tail_t)], sem) if tail_t else None
                is_full = start + BLK <= N
                is_part = jnp.logical_and(start < N, start + BLK > N)
                return start, buf, full, part, is_full, is_part

            def start_load(j):
                start, buf, full, part, is_full, is_part = descs(j)
                pl.when(is_full)(full.start)
                if part is not None:
                    pl.when(is_part)(part.start)

            def wait_load(j):
                start, buf, full, part, is_full, is_part = descs(j)
                pl.when(is_full)(full.wait)
                if part is not None:
                    pl.when(is_part)(part.wait)

            start_load(0)
            for j in range(tb_per_w):
                if j + 1 < tb_per_w:
                    start_load(j + 1)
                start, buf, _, _, _, _ = descs(j)
                wait_load(j)

                @pl.when(start < N)
                def _scatter():
                    pltpu.sync_copy(buf, acc.at[ids_v.at[j]], add=True)

        _zero_acc(acc, zbuf, sid)
        plsc.subcore_barrier()
        seg_sum_phase(tree_hbm)
        plsc.subcore_barrier()
        _writeout(acc, out_t, cid, sid)
        plsc.subcore_barrier()

        _zero_acc(acc, zbuf, sid)
        plsc.subcore_barrier()
        seg_sum_phase(inter_hbm)
        plsc.subcore_barrier()
        _writeout(acc, out_i, cid, sid)
        plsc.subcore_barrier()

        _zero_acc(acc, zbuf, sid)
        plsc.subcore_barrier()
        def a_desc(j):
            buf, sem = bufs[j % 2], sems[j % 2]
            return buf, pltpu.make_async_copy(
                graph_hbm.at[ai_v.at[j]], buf, sem)

        def a_start(j):
            astart = (wid * ab_per_w + j) * BLK
            _, d = a_desc(j)
            pl.when(astart < A)(d.start)

        a_start(0)
        for j in range(ab_per_w):
            astart = (wid * ab_per_w + j) * BLK
            if j + 1 < ab_per_w:
                a_start(j + 1)
            buf, d = a_desc(j)
            pl.when(astart < A)(d.wait)

            @pl.when(astart < A)
            def _scatter_a():
                pltpu.sync_copy(buf, acc.at[as_v.at[j]], add=True)
        plsc.subcore_barrier()
        _writeout(acc, out_g, cid, sid)

    return body(tree_vecs, inter_vecs, graph_vecs, tseg3d, aidx3d, aseg3d)


def _sc_counts(tseg3d, n_nodes, tb_per_w):
    N = n_nodes
    mesh = plsc.VectorSubcoreMesh(**_MESH)

    @functools.partial(
        pl.kernel,
        mesh=mesh,
        out_type=jax.ShapeDtypeStruct((NC, B, H), jnp.float32),
        scratch_types=[
            pltpu.VMEM((tb_per_w, BLK), jnp.int32),
            pltpu.VMEM((BLK, H), jnp.float32),
            pltpu.VMEM((BLK, H), jnp.float32),
            pltpu.VMEM_SHARED((R, H), jnp.float32),
        ],
    )
    def body(tseg_hbm, out_c, ids_v, ones_v, zbuf, acc):
        cid = lax.axis_index("c")
        sid = lax.axis_index("s")
        wid = sid * NC + cid

        _fill_rows(zbuf, 0.0)
        _fill_rows(ones_v, 1.0)
        pltpu.sync_copy(tseg_hbm.at[wid], ids_v)

        _zero_acc(acc, zbuf, sid)
        plsc.subcore_barrier()
        for j in range(tb_per_w):
            start = (wid * tb_per_w + j) * BLK

            @pl.when(start < N)
            def _blk():
                pltpu.sync_copy(ones_v, acc.at[ids_v.at[j]], add=True)
        plsc.subcore_barrier()
        _writeout(acc, out_c, cid, sid)

    return body(tseg3d)


def _tc_heads_body(root_ref, t_ref, i_ref, g_ref, c_ref,
                   wt_ref, bt_ref, wg_ref, bg_ref, out_t_ref, out_g_ref):
    t = t_ref[0] + t_ref[1]
    im = i_ref[0] + i_ref[1]
    g = g_ref[0] + g_ref[1]
    c = c_ref[0, :, 0:1] + c_ref[1, :, 0:1]
    inv = 1.0 / jnp.maximum(c, 1.0)
    wt = wt_ref[...]
    dot = functools.partial(jnp.dot, preferred_element_type=jnp.float32)
    out = dot(root_ref[...], wt[0:H])
    out += dot(t * inv, wt[H:2 * H])
    out += dot(im * inv, wt[2 * H:3 * H])
    out_t_ref[...] = out + bt_ref[...]
    out_g_ref[...] = dot(g, wg_ref[...]) + bg_ref[...]


def _tc_heads(root_vecs, t_part, i_part, g_part, c_part, W_T, b_T, W_G, b_G):
    bsz = 512
    grid = (B // bsz,)
    return pl.pallas_call(
        _tc_heads_body,
        grid=grid,
        in_specs=[
            pl.BlockSpec((bsz, H), lambda i: (i, 0)),
            pl.BlockSpec((NC, bsz, H), lambda i: (0, i, 0)),
            pl.BlockSpec((NC, bsz, H), lambda i: (0, i, 0)),
            pl.BlockSpec((NC, bsz, H), lambda i: (0, i, 0)),
            pl.BlockSpec((NC, bsz, H), lambda i: (0, i, 0)),
            pl.BlockSpec((3 * H, L), lambda i: (0, 0)),
            pl.BlockSpec((1, L), lambda i: (0, 0)),
            pl.BlockSpec((H, L), lambda i: (0, 0)),
            pl.BlockSpec((1, L), lambda i: (0, 0)),
        ],
        out_specs=[
            pl.BlockSpec((bsz, L), lambda i: (i, 0)),
            pl.BlockSpec((bsz, L), lambda i: (i, 0)),
        ],
        out_shape=[
            jax.ShapeDtypeStruct((B, L), jnp.float32),
            jax.ShapeDtypeStruct((B, L), jnp.float32),
        ],
    )(root_vecs, t_part, i_part, g_part, c_part, W_T, b_T, W_G, b_G)


def kernel(root_vecs, tree_vecs, inter_vecs, graph_vecs, tree_segment_ids,
           anchor_idx, anchor_segment_ids, W_T, b_T, W_G, b_G):
    N = tree_vecs.shape[0]
    A = anchor_idx.shape[0]

    tb_per_w = -(-N // (NW * BLK))
    ab_per_w = -(-A // (NW * BLK))
    n_pad = NW * tb_per_w * BLK - N
    a_pad = NW * ab_per_w * BLK - A

    tseg = tree_segment_ids.astype(jnp.int32)
    tseg = jnp.concatenate(
        [tseg, B + (jnp.arange(n_pad, dtype=jnp.int32) % DUMMY)])
    tseg3d = tseg.reshape(NW, tb_per_w, BLK)

    aidx = anchor_idx.astype(jnp.int32)
    aidx = jnp.concatenate(
        [aidx, (jnp.arange(a_pad, dtype=jnp.int32) * 997) % N])
    aidx3d = aidx.reshape(NW, ab_per_w, BLK)

    aseg = anchor_segment_ids.astype(jnp.int32)
    aseg = jnp.concatenate(
        [aseg, B + (jnp.arange(a_pad, dtype=jnp.int32) % DUMMY)])
    aseg3d = aseg.reshape(NW, ab_per_w, BLK)

    t_part, i_part, g_part = _sc_pool(
        tree_vecs, inter_vecs, graph_vecs, tseg3d, aidx3d, aseg3d,
        N, A, tb_per_w, ab_per_w)
    c_part = _sc_counts(tseg3d, N, tb_per_w)

    tree_out, graph_out = _tc_heads(
        root_vecs, t_part, i_part, g_part, c_part,
        W_T, b_T.reshape(1, L), W_G, b_G.reshape(1, L))
    return (tree_out, tree_out, graph_out)
